# Initial kernel scaffold; baseline (speedup 1.0000x reference)
import jax
import jax.numpy as jnp
from jax.experimental import pallas as pl


def kernel(x, edge_index, edge_attr, params):
    raise NotImplementedError("write your pallas kernel here")


# speedup vs baseline: 2.6988x; 2.6988x over previous
import functools

import jax
import jax.numpy as jnp
from jax import lax
from jax.experimental import pallas as pl
from jax.experimental.pallas import tpu as pltpu
from jax.experimental.pallas import tpu_sc as plsc

N = 10000
E = 320000
H = 128
EPS = 1e-5

NC, NS = 2, 16
NW = NC * NS
PERW = E // NW
KI = 80
JG = 5
GRP = KI * JG
NG = PERW // GRP
NPT = 624
NTAIL = N - NS * NPT
SKI = 40
SJG = 5
SGRP = SKI * SJG
SNG = PERW // SGRP

BE = 1600
GEDGE = E // BE
NB = 2000
GNODE = N // NB

F32 = jnp.float32


def _mesh():
    return plsc.VectorSubcoreMesh(core_axis_name="c", subcore_axis_name="s",
                                  num_cores=NC, num_subcores=NS)


@functools.partial(
    pl.kernel,
    out_type=(jax.ShapeDtypeStruct((E, H), F32),
              jax.ShapeDtypeStruct((E, H), F32)),
    mesh=_mesh(),
    scratch_types=[
        pltpu.VMEM((JG, KI), jnp.int32),
        pltpu.VMEM((JG, KI), jnp.int32),
        pltpu.VMEM((GRP, H), F32),
        pltpu.VMEM((GRP, H), F32),
        pltpu.SemaphoreType.DMA,
        pltpu.SemaphoreType.DMA,
    ],
)
def _sc_gather(table, srcr, dstr, hs_out, hd_out,
               idx_s, idx_d, row_s, row_d, sem_s, sem_d):
    c = lax.axis_index("c")
    s = lax.axis_index("s")
    w = s * NC + c

    def body(g, carry):
        base = w * PERW + g * GRP
        pltpu.sync_copy(srcr.at[w, g], idx_s)
        pltpu.sync_copy(dstr.at[w, g], idx_d)
        cps = []
        for j in range(JG):
            cps.append(pltpu.async_copy(
                table.at[idx_s.at[j]], row_s.at[pl.ds(j * KI, KI)], sem_s))
            cps.append(pltpu.async_copy(
                table.at[idx_d.at[j]], row_d.at[pl.ds(j * KI, KI)], sem_d))
        for cp in cps:
            cp.wait()
        pltpu.sync_copy(row_s, hs_out.at[pl.ds(base, GRP)])
        pltpu.sync_copy(row_d, hd_out.at[pl.ds(base, GRP)])
        return carry

    lax.fori_loop(0, NG, body, 0)


@functools.partial(
    pl.kernel,
    out_type=jax.ShapeDtypeStruct((NC, N, H), F32),
    mesh=_mesh(),
    scratch_types=[
        pltpu.VMEM_SHARED((N, H), F32),
        pltpu.VMEM((SGRP, H), F32),
        pltpu.VMEM((SJG, SKI), jnp.int32),
    ],
)
def _sc_scatter(msg, dstr, zeros, part, acc, rows, idx):
    c = lax.axis_index("c")
    s = lax.axis_index("s")
    w = s * NC + c

    pltpu.sync_copy(zeros.at[pl.ds(s * NPT, NPT)], acc.at[pl.ds(s * NPT, NPT)])

    @pl.when(s == NS - 1)
    def _():
        pltpu.sync_copy(zeros.at[pl.ds(NS * NPT, NTAIL)],
                        acc.at[pl.ds(NS * NPT, NTAIL)])

    plsc.subcore_barrier()

    def body(g, carry):
        base = w * PERW + g * SGRP
        pltpu.sync_copy(msg.at[pl.ds(base, SGRP)], rows)
        pltpu.sync_copy(dstr.at[w, g], idx)
        for j in range(SJG):
            pltpu.sync_copy(rows.at[pl.ds(j * SKI, SKI)], acc.at[idx.at[j]],
                            add=True)
        return carry

    lax.fori_loop(0, SNG, body, 0)
    plsc.subcore_barrier()
    pltpu.sync_copy(acc.at[pl.ds(s * NPT, NPT)],
                    part.at[c, pl.ds(s * NPT, NPT)])

    @pl.when(s == NS - 1)
    def _():
        pltpu.sync_copy(acc.at[pl.ds(NS * NPT, NTAIL)],
                        part.at[c, pl.ds(NS * NPT, NTAIL)])


def _dot(a, b):
    return jnp.dot(a, b, preferred_element_type=F32)


def _relu(v):
    return jnp.maximum(v, 0.0)


def _rep(shape):
    return pl.BlockSpec(shape, lambda i: tuple(0 for _ in shape))


def _blk(shape):
    return pl.BlockSpec(shape, lambda i: (i,) + tuple(0 for _ in shape[1:]))


def _tc_params():
    return pltpu.CompilerParams(dimension_semantics=("arbitrary",))


def _proj_body(x_ref, w_ref, b_ref, o_ref):
    o_ref[...] = _relu(_dot(x_ref[...], w_ref[...]) + b_ref[...])


def _proj(x, w, b):
    return pl.pallas_call(
        _proj_body,
        grid=(GNODE,),
        in_specs=[_blk((NB, H)), _rep((H, H)), _rep((1, H))],
        out_specs=_blk((NB, H)),
        out_shape=jax.ShapeDtypeStruct((N, H), F32),
        compiler_params=_tc_params(),
    )(x, w, b)


def _edge_layer(mode, e_srcs, wts):
    n_esrc = len(e_srcs)

    def body(*refs):
        if mode == 0:
            (ea_ref, hs_ref, hd_ref,
             we, be, w1s, w1d, w1e, b1, w2, b2,
             v1d, v1s, v1e, c1, v2, c2, v3, c3,
             d_ref, m_ref, s1_ref, s2_ref) = refs
            e_in = _relu(_dot(ea_ref[...], we[...]) + be[...])
        elif mode == 1:
            (dp_ref, hs_ref, hd_ref, sc, sh,
             w1s, w1d, w1e, b1, w2, b2,
             v1d, v1s, v1e, c1, v2, c2, v3, c3,
             e_ref, d_ref, m_ref, s1_ref, s2_ref) = refs
            e_in = _relu(dp_ref[...] * sc[...] + sh[...])
            e_ref[...] = e_in
        else:
            (eb_ref, dp_ref, hs_ref, hd_ref, sc, sh,
             w1s, w1d, w1e, b1, w2, b2,
             v1d, v1s, v1e, c1, v2, c2, v3, c3,
             e_ref, d_ref, m_ref, s1_ref, s2_ref) = refs
            e_in = eb_ref[...] + _relu(dp_ref[...] * sc[...] + sh[...])
            e_ref[...] = e_in
        hsv = hs_ref[...]
        hdv = hd_ref[...]
        t = _relu(_dot(hsv, w1s[...]) + _dot(hdv, w1d[...])
                  + _dot(e_in, w1e[...]) + b1[...])
        d = _dot(t, w2[...]) + b2[...]
        d_ref[...] = d
        m = _relu(_dot(hdv, v1d[...]) + _dot(hsv, v1s[...])
                  + _dot(d, v1e[...]) + c1[...])
        m = _relu(_dot(m, v2[...]) + c2[...])
        m_ref[...] = _dot(m, v3[...]) + c3[...]

        @pl.when(pl.program_id(0) == 0)
        def _():
            s1_ref[...] = jnp.zeros_like(s1_ref)
            s2_ref[...] = jnp.zeros_like(s2_ref)

        s1_ref[...] += jnp.sum(d, axis=0, keepdims=True)
        s2_ref[...] += jnp.sum(d * d, axis=0, keepdims=True)

    esrc_specs = {0: [_blk((BE, 16)), _blk((BE, H)), _blk((BE, H))],
                  1: [_blk((BE, H))] * 3 + [_rep((1, H))] * 2,
                  2: [_blk((BE, H))] * 4 + [_rep((1, H))] * 2}[mode]
    w_specs = [_rep(w.shape) for w in wts]
    out_shapes = [jax.ShapeDtypeStruct((E, H), F32),
                  jax.ShapeDtypeStruct((E, H), F32),
                  jax.ShapeDtypeStruct((1, H), F32),
                  jax.ShapeDtypeStruct((1, H), F32)]
    out_specs = [_blk((BE, H)), _blk((BE, H)), _rep((1, H)), _rep((1, H))]
    if mode != 0:
        out_shapes = [jax.ShapeDtypeStruct((E, H), F32)] + out_shapes
        out_specs = [_blk((BE, H))] + out_specs
    return pl.pallas_call(
        body,
        grid=(GEDGE,),
        in_specs=esrc_specs + w_specs,
        out_specs=out_specs,
        out_shape=out_shapes,
        compiler_params=_tc_params(),
    )(*e_srcs, *wts)


def _node_mlp(h, part, w1h, w1a, b1, w2, b2):
    def body(h_ref, p_ref, w1h_r, w1a_r, b1_r, w2_r, b2_r,
             r_ref, s1_ref, s2_ref):
        agg = p_ref[0] + p_ref[1]
        u = _relu(_dot(h_ref[...], w1h_r[...]) + _dot(agg, w1a_r[...])
                  + b1_r[...])
        r = _dot(u, w2_r[...]) + b2_r[...]
        r_ref[...] = r

        @pl.when(pl.program_id(0) == 0)
        def _():
            s1_ref[...] = jnp.zeros_like(s1_ref)
            s2_ref[...] = jnp.zeros_like(s2_ref)

        s1_ref[...] += jnp.sum(r, axis=0, keepdims=True)
        s2_ref[...] += jnp.sum(r * r, axis=0, keepdims=True)

    return pl.pallas_call(
        body,
        grid=(GNODE,),
        in_specs=[_blk((NB, H)),
                  pl.BlockSpec((NC, NB, H), lambda i: (0, i, 0)),
                  _rep((H, H)), _rep((H, H)), _rep((1, H)),
                  _rep((H, H)), _rep((1, H))],
        out_specs=[_blk((NB, H)), _rep((1, H)), _rep((1, H))],
        out_shape=[jax.ShapeDtypeStruct((N, H), F32),
                   jax.ShapeDtypeStruct((1, H), F32),
                   jax.ShapeDtypeStruct((1, H), F32)],
        compiler_params=_tc_params(),
    )(h, part, w1h, w1a, b1, w2, b2)


def _node_bn_apply(r, h, sc, sh, residual):
    def body(r_ref, h_ref, sc_r, sh_r, o_ref):
        v = _relu(r_ref[...] * sc_r[...] + sh_r[...])
        if residual:
            v = h_ref[...] + v
        o_ref[...] = v

    return pl.pallas_call(
        body,
        grid=(GNODE,),
        in_specs=[_blk((NB, H)), _blk((NB, H)), _rep((1, H)), _rep((1, H))],
        out_specs=_blk((NB, H)),
        out_shape=jax.ShapeDtypeStruct((N, H), F32),
        compiler_params=_tc_params(),
    )(r, h, sc, sh)


def _edge_readout(ebase, dprev, sc, sh, a1, a1b, a2, a2b):
    def body(eb_ref, dp_ref, sc_r, sh_r, a1_r, a1b_r, a2_r, a2b_r,
             e_ref, at_ref):
        e3 = eb_ref[...] + _relu(dp_ref[...] * sc_r[...] + sh_r[...])
        e_ref[...] = e3
        t = _relu(_dot(e3, a1_r[...]) + a1b_r[...])
        logit = _dot(t, a2_r[...]) + a2b_r[...]
        at_ref[...] = jax.nn.sigmoid(logit)

    return pl.pallas_call(
        body,
        grid=(GEDGE,),
        in_specs=[_blk((BE, H)), _blk((BE, H)), _rep((1, H)), _rep((1, H)),
                  _rep((H, 64)), _rep((1, 64)), _rep((64, 1)), _rep((1, 1))],
        out_specs=[_blk((BE, H)), _blk((BE, 1))],
        out_shape=[jax.ShapeDtypeStruct((E, H), F32),
                   jax.ShapeDtypeStruct((E, 1), F32)],
        compiler_params=_tc_params(),
    )(ebase, dprev, sc, sh, a1, a1b, a2, a2b)


def _node_readout(h3, a1, a1b, a2, a2b, t1, t1b, t2, t2b, wo, bo):
    def body(h_ref, a1_r, a1b_r, a2_r, a2b_r, t1_r, t1b_r, t2_r, t2b_r,
             wo_r, bo_r, ge_ref, at_ref, tt_ref):
        h = h_ref[...]
        la = _dot(_relu(_dot(h, a1_r[...]) + a1b_r[...]), a2_r[...]) + a2b_r[...]
        m = jnp.max(la, axis=0, keepdims=True)
        p = jnp.exp(la - m)
        attn = p / jnp.sum(p, axis=0, keepdims=True)
        at_ref[...] = attn
        lt = _dot(_relu(_dot(h, t1_r[...]) + t1b_r[...]), t2_r[...]) + t2b_r[...]
        tt_ref[...] = jax.nn.sigmoid(lt)
        ge = jnp.sum(h * attn, axis=0, keepdims=True)
        ge_ref[...] = _dot(ge, wo_r[...]) + bo_r[...]

    return pl.pallas_call(
        body,
        grid=(1,),
        in_specs=[_rep((N, H)),
                  _rep((H, 64)), _rep((1, 64)), _rep((64, 1)), _rep((1, 1)),
                  _rep((H, 64)), _rep((1, 64)), _rep((64, 1)), _rep((1, 1)),
                  _rep((H, H)), _rep((1, H))],
        out_specs=[_rep((1, H)), _rep((N, 1)), _rep((N, 1))],
        out_shape=[jax.ShapeDtypeStruct((1, H), F32),
                   jax.ShapeDtypeStruct((N, 1), F32),
                   jax.ShapeDtypeStruct((N, 1), F32)],
        compiler_params=_tc_params(),
    )(h3, a1, a1b, a2, a2b, t1, t1b, t2, t2b, wo, bo)


def _row(v):
    return v.reshape(1, -1)


def _bn_scale_shift(s1, s2, g, b):
    mean = s1 / E
    var = s2 / E - mean * mean
    scale = _row(g) / jnp.sqrt(var + EPS)
    shift = _row(b) - mean * scale
    return scale, shift


def kernel(x, edge_index, edge_attr, params):
    src = edge_index[0].astype(jnp.int32)
    dst = edge_index[1].astype(jnp.int32)
    srcr = src.reshape(NW, NG, JG, KI)
    dstr = dst.reshape(NW, NG, JG, KI)
    dstr_s = dst.reshape(NW, SNG, SJG, SKI)
    zeros = jnp.zeros((N, H), F32)

    h = _proj(x, params["node_in"]["w"], _row(params["node_in"]["b"]))

    ebase = dprev = e_sc = e_sh = None
    for i, lp in enumerate(params["layers"]):
        hs, hd = _sc_gather(h, srcr, dstr)
        w1 = lp["edge_upd"][0]["w"]
        v1 = lp["edge_mlp"][0]["w"]
        wts = [w1[:H], w1[H:2 * H], w1[2 * H:], _row(lp["edge_upd"][0]["b"]),
               lp["edge_upd"][1]["w"], _row(lp["edge_upd"][1]["b"]),
               v1[:H], v1[H:2 * H], v1[2 * H:], _row(lp["edge_mlp"][0]["b"]),
               lp["edge_mlp"][1]["w"], _row(lp["edge_mlp"][1]["b"]),
               lp["edge_mlp"][2]["w"], _row(lp["edge_mlp"][2]["b"])]
        if i == 0:
            ea = edge_attr
            wts = [params["edge_in"]["w"], _row(params["edge_in"]["b"])] + wts
            d, msg, s1, s2 = _edge_layer(0, [ea, hs, hd], wts)
        elif i == 1:
            e_out, d, msg, s1, s2 = _edge_layer(
                1, [dprev, hs, hd, e_sc, e_sh], wts)
            ebase = e_out
        else:
            e_out, d, msg, s1, s2 = _edge_layer(
                2, [ebase, dprev, hs, hd, e_sc, e_sh], wts)
            ebase = e_out
        dprev = d
        e_sc, e_sh = _bn_scale_shift(s1, s2, lp["bn_edge"]["g"],
                                     lp["bn_edge"]["b"])

        part = _sc_scatter(msg, dstr_s, zeros)
        nw1 = lp["node_mlp"][0]["w"]
        r, ns1, ns2 = _node_mlp(h, part, nw1[:H], nw1[H:],
                                _row(lp["node_mlp"][0]["b"]),
                                lp["node_mlp"][1]["w"],
                                _row(lp["node_mlp"][1]["b"]))
        mean = ns1 / N
        var = ns2 / N - mean * mean
        n_sc = _row(lp["bn_node"]["g"]) / jnp.sqrt(var + EPS)
        n_sh = _row(lp["bn_node"]["b"]) - mean * n_sc
        h = _node_bn_apply(r, h, n_sc, n_sh, residual=(i > 0))

    e3, eattn = _edge_readout(
        ebase, dprev, e_sc, e_sh,
        params["edge_attn"][0]["w"], _row(params["edge_attn"][0]["b"]),
        params["edge_attn"][1]["w"], _row(params["edge_attn"][1]["b"]))
    ge, nattn, taint = _node_readout(
        h,
        params["node_attn"][0]["w"], _row(params["node_attn"][0]["b"]),
        params["node_attn"][1]["w"], _row(params["node_attn"][1]["b"]),
        params["taint"][0]["w"], _row(params["taint"][0]["b"]),
        params["taint"][1]["w"], _row(params["taint"][1]["b"]),
        params["out_proj"]["w"], _row(params["out_proj"]["b"]))
    return (ge, h, e3, nattn, eattn, taint)
